# baseline (speedup 1.0000x reference)
import functools

import jax
import jax.numpy as jnp
from jax import lax
from jax.experimental import pallas as pl
from jax.experimental.pallas import tpu as pltpu
from jax.experimental.pallas import tpu_sc as plsc

F32 = jnp.float32
B, N, H, M = 8, 64, 256, 128
NN = B * N
NC, NS = 2, 16
NW = NC * NS


def _silu(x):
    return x * jax.nn.sigmoid(x)


def _prep_body(hf, srcf, dstf, ws, wd, tab, wv, nnb):
    tab[:NN, :] = jnp.dot(hf[...], ws[...], preferred_element_type=F32)
    tab[NN:, :] = jnp.dot(hf[...], wd[...], preferred_element_type=F32)
    e = srcf.shape[0]
    iota = lax.broadcasted_iota(jnp.int32, (e, N), 1)
    os_ = (srcf[...] == iota).astype(F32)
    od_ = (dstf[...] == iota).astype(F32)
    cnt = lax.dot_general(os_, od_, (((0,), (0,)), ((), ())),
                          preferred_element_type=F32)
    mult = jnp.sum(jnp.dot(os_, cnt, preferred_element_type=F32) * od_,
                   axis=1, keepdims=True)
    wv[...] = 1.0 / mult
    nnb[...] = jnp.sum(cnt, axis=1, keepdims=True)


def _edge_body(gs, gd, srcf, dstf, cpad, hf, wv, nnb, wn, bm1, wm2, bm2,
               wc1, bc1, wc2r, wh1h, wh1s, bh1, wh2, bh2, co, ho):
    e = srcf.shape[0]
    iota = lax.broadcasted_iota(jnp.int32, (e, N), 1)
    os_ = (srcf[...] == iota).astype(F32)
    od_ = (dstf[...] == iota).astype(F32)
    cp = cpad[...]
    diff = (jnp.dot(os_, cp, preferred_element_type=F32)
            - jnp.dot(od_, cp, preferred_element_type=F32))
    d2 = jnp.sum(diff * diff, axis=1, keepdims=True)
    n2 = jnp.sqrt(d2)
    g = gs[...] + gd[...]
    m = _silu(g + n2 * wn[...] + bm1[...])
    f = _silu(jnp.dot(m, wm2[...], preferred_element_type=F32) + bm2[...])
    cq = _silu(jnp.dot(f, wc1[...], preferred_element_type=F32) + bc1[...])
    c = jnp.sum(cq * wc2r[...], axis=1, keepdims=True)
    w = wv[...]
    sum_h = lax.dot_general(od_, f * w, (((0,), (0,)), ((), ())),
                            preferred_element_type=F32)
    sum_t = lax.dot_general(od_, diff * (c * w), (((0,), (0,)), ((), ())),
                            preferred_element_type=F32)
    co[...] = cp + sum_t / nnb[...]
    pre = _silu(jnp.dot(hf[...], wh1h[...], preferred_element_type=F32)
                + jnp.dot(sum_h, wh1s[...], preferred_element_type=F32)
                + bh1[...])
    ho[...] = jnp.dot(pre, wh2[...], preferred_element_type=F32) + bh2[...]


def _mesh():
    return plsc.VectorSubcoreMesh(core_axis_name="c", subcore_axis_name="s",
                                  num_cores=NC, num_subcores=NS)


def _sc_gather(tab, idxr, ef):
    chunk = ef // NW
    nj = chunk // 128

    @functools.partial(
        pl.kernel, mesh=_mesh(),
        out_type=(jax.ShapeDtypeStruct((ef, M), F32),
                  jax.ShapeDtypeStruct((ef, M), F32)),
        scratch_types=[pltpu.VMEM((2 * nj, 128), jnp.int32),
                       pltpu.VMEM((chunk, M), F32),
                       pltpu.VMEM((chunk, M), F32),
                       pltpu.SemaphoreType.DMA,
                       pltpu.SemaphoreType.DMA,
                       pltpu.SemaphoreType.DMA,
                       pltpu.SemaphoreType.DMA,
                       pltpu.SemaphoreType.DMA],
    )
    def k(tab_hbm, idx_hbm, gs_hbm, gd_hbm,
          idx, bufs, bufd, sem0, sem1, sem2, sem3, wsem):
        cid = lax.axis_index("c")
        sid = lax.axis_index("s")
        wid = sid * NC + cid
        base = wid * chunk
        pltpu.sync_copy(idx_hbm.at[wid], idx)
        gsems = [sem0, sem1, sem2, sem3]
        gets = [pltpu.async_copy(tab_hbm.at[idx.at[j]],
                                 (bufs if j < nj else bufd)
                                 .at[pl.ds((j % nj) * 128, 128)],
                                 gsems[j])
                for j in range(2 * nj)]
        puts = []
        for j in range(2 * nj):
            gets[j].wait()
            dst = gs_hbm if j < nj else gd_hbm
            buf = bufs if j < nj else bufd
            puts.append(pltpu.async_copy(
                buf.at[pl.ds((j % nj) * 128, 128)],
                dst.at[pl.ds(base + (j % nj) * 128, 128)], wsem))
        for p in puts:
            p.wait()

    return k(tab, idxr)


def kernel(coords, hidden, edges, Wm1, bm1, Wm2, bm2, Wc1, bc1, Wc2,
           Wh1, bh1, Wh2, bh2):
    e = edges.shape[1]
    ef = B * e
    cf = coords.reshape(NN, 3).astype(F32)
    hf = hidden.reshape(NN, H).astype(F32)
    cpad = jnp.concatenate([cf, jnp.zeros((NN, 125), F32)], axis=1)

    srcf = edges[0].astype(jnp.int32)[:, None]
    dstf = edges[1].astype(jnp.int32)[:, None]

    tab, wv, nnb = pl.pallas_call(
        _prep_body,
        out_shape=(jax.ShapeDtypeStruct((2 * NN, M), F32),
                   jax.ShapeDtypeStruct((e, 1), F32),
                   jax.ShapeDtypeStruct((N, 1), F32)),
    )(hf, srcf, dstf, Wm1[1:1 + H], Wm1[1 + H:])

    offs = (jnp.arange(B, dtype=jnp.int32) * N)[:, None]
    nj = ef // NW // 128
    src_flat = (edges[0][None, :] + offs).reshape(NW, nj, 128)
    dst_flat = (edges[1][None, :] + offs + NN).reshape(NW, nj, 128)
    idx_all = jnp.concatenate([src_flat, dst_flat], axis=1)

    gs, gd = _sc_gather(tab, idx_all, ef)

    full = lambda shape: pl.BlockSpec(shape, lambda i: tuple(0 for _ in shape))
    co, ho = pl.pallas_call(
        _edge_body,
        grid=(B,),
        in_specs=[pl.BlockSpec((e, M), lambda i: (i, 0)),
                  pl.BlockSpec((e, M), lambda i: (i, 0)),
                  full((e, 1)), full((e, 1)),
                  pl.BlockSpec((N, 128), lambda i: (i, 0)),
                  pl.BlockSpec((N, H), lambda i: (i, 0)),
                  full((e, 1)), full((N, 1)),
                  full((1, M)), full((1, M)), full((M, M)), full((1, M)),
                  full((M, M)), full((1, M)), full((1, M)),
                  full((H, M)), full((M, M)), full((1, M)),
                  full((M, H)), full((1, H))],
        out_specs=(pl.BlockSpec((N, 128), lambda i: (i, 0)),
                   pl.BlockSpec((N, H), lambda i: (i, 0))),
        out_shape=(jax.ShapeDtypeStruct((NN, 128), F32),
                   jax.ShapeDtypeStruct((NN, H), F32)),
    )(gs, gd, srcf, dstf, cpad, hf, wv, nnb, Wm1[0:1], bm1[None, :], Wm2,
      bm2[None, :], Wc1, bc1[None, :], Wc2.reshape(1, M),
      Wh1[:H], Wh1[H:], bh1[None, :], Wh2, bh2[None, :])

    coords_out = co[:, :3].reshape(B, N, 3)
    hidden_out = ho.reshape(B, N, H)
    return coords_out, hidden_out


# speedup vs baseline: 1.0285x; 1.0285x over previous
import functools

import jax
import jax.numpy as jnp
from jax import lax
from jax.experimental import pallas as pl
from jax.experimental.pallas import tpu as pltpu
from jax.experimental.pallas import tpu_sc as plsc

F32 = jnp.float32
B, N, H, M = 8, 64, 256, 128
NN = B * N
NC, NS = 2, 16
NW = NC * NS


def _silu(x):
    return x * jax.nn.sigmoid(x)


def _prep_body(hf, srcf, dstf, srcr, dstr, wm1, tab, wv, nnb, idxcat):
    tab[:NN, :] = jnp.dot(hf[...], wm1[1:1 + H, :],
                          preferred_element_type=F32)
    tab[NN:, :] = jnp.dot(hf[...], wm1[1 + H:, :],
                          preferred_element_type=F32)
    e = srcf.shape[0]
    iota = lax.broadcasted_iota(jnp.int32, (e, N), 1)
    os_ = (srcf[...] == iota).astype(F32)
    od_ = (dstf[...] == iota).astype(F32)
    cnt = lax.dot_general(os_, od_, (((0,), (0,)), ((), ())),
                          preferred_element_type=F32)
    mult = jnp.sum(jnp.dot(os_, cnt, preferred_element_type=F32) * od_,
                   axis=1, keepdims=True)
    wv[...] = 1.0 / mult
    nnb[...] = jnp.sum(cnt, axis=1, keepdims=True)
    rows_s = jnp.concatenate([srcr[...]] * B, axis=0)
    rows_d = jnp.concatenate([dstr[...]] * B, axis=0)
    er = e // 128
    boff = (lax.broadcasted_iota(jnp.int32, (B * er, 128), 0) // er) * N
    idxcat[:B * er, :] = rows_s + boff
    idxcat[B * er:, :] = rows_d + boff + NN


def _edge_body(gs, gd, srcf, dstf, cf, hf, wv, nnb, wn, bm1, wm2, bm2,
               wc1, bc1, wc2r, wh1h, wh1s, bh1, wh2, bh2, co, ho):
    e = srcf.shape[0]
    iota = lax.broadcasted_iota(jnp.int32, (e, N), 1)
    os_ = (srcf[...] == iota).astype(F32)
    od_ = (dstf[...] == iota).astype(F32)
    cp = jnp.concatenate([cf[...], jnp.zeros((N, 125), F32)], axis=1)
    diff = (jnp.dot(os_, cp, preferred_element_type=F32)
            - jnp.dot(od_, cp, preferred_element_type=F32))
    d2 = jnp.sum(diff * diff, axis=1, keepdims=True)
    n2 = jnp.sqrt(d2)
    g = gs[...] + gd[...]
    m = _silu(g + n2 * wn[...] + bm1[...])
    f = _silu(jnp.dot(m, wm2[...], preferred_element_type=F32) + bm2[...])
    cq = _silu(jnp.dot(f, wc1[...], preferred_element_type=F32) + bc1[...])
    c = jnp.sum(cq * wc2r[...], axis=1, keepdims=True)
    w = wv[...]
    sum_h = lax.dot_general(od_, f * w, (((0,), (0,)), ((), ())),
                            preferred_element_type=F32)
    sum_t = lax.dot_general(od_, diff * (c * w), (((0,), (0,)), ((), ())),
                            preferred_element_type=F32)
    co[...] = cf[...] + sum_t[:, 0:3] / nnb[...]
    pre = _silu(jnp.dot(hf[...], wh1h[...], preferred_element_type=F32)
                + jnp.dot(sum_h, wh1s[...], preferred_element_type=F32)
                + bh1[...])
    ho[...] = jnp.dot(pre, wh2[...], preferred_element_type=F32) + bh2[...]


def _mesh():
    return plsc.VectorSubcoreMesh(core_axis_name="c", subcore_axis_name="s",
                                  num_cores=NC, num_subcores=NS)


def _sc_gather(tab, idxcat, ef):
    chunk = ef // NW
    nj = chunk // 128
    nr = ef // 128

    @functools.partial(
        pl.kernel, mesh=_mesh(),
        out_type=(jax.ShapeDtypeStruct((ef, M), F32),
                  jax.ShapeDtypeStruct((ef, M), F32)),
        scratch_types=[pltpu.VMEM((2 * nj, 128), jnp.int32),
                       pltpu.VMEM((chunk, M), F32),
                       pltpu.VMEM((chunk, M), F32),
                       pltpu.SemaphoreType.DMA,
                       pltpu.SemaphoreType.DMA,
                       pltpu.SemaphoreType.DMA,
                       pltpu.SemaphoreType.DMA,
                       pltpu.SemaphoreType.DMA],
    )
    def k(tab_hbm, idx_hbm, gs_hbm, gd_hbm,
          idx, bufs, bufd, sem0, sem1, sem2, sem3, wsem):
        cid = lax.axis_index("c")
        sid = lax.axis_index("s")
        wid = sid * NC + cid
        base = wid * chunk
        pltpu.sync_copy(idx_hbm.at[pl.ds(nj * wid, nj)],
                        idx.at[pl.ds(0, nj)])
        pltpu.sync_copy(idx_hbm.at[pl.ds(nr + nj * wid, nj)],
                        idx.at[pl.ds(nj, nj)])
        gsems = [sem0, sem1, sem2, sem3]
        gets = [pltpu.async_copy(tab_hbm.at[idx.at[j]],
                                 (bufs if j < nj else bufd)
                                 .at[pl.ds((j % nj) * 128, 128)],
                                 gsems[j])
                for j in range(2 * nj)]
        puts = []
        for j in range(2 * nj):
            gets[j].wait()
            dst = gs_hbm if j < nj else gd_hbm
            buf = bufs if j < nj else bufd
            puts.append(pltpu.async_copy(
                buf.at[pl.ds((j % nj) * 128, 128)],
                dst.at[pl.ds(base + (j % nj) * 128, 128)], wsem))
        for p in puts:
            p.wait()

    return k(tab, idxcat)


def kernel(coords, hidden, edges, Wm1, bm1, Wm2, bm2, Wc1, bc1, Wc2,
           Wh1, bh1, Wh2, bh2):
    e = edges.shape[1]
    ef = B * e
    cf = coords.reshape(NN, 3).astype(F32)
    hf = hidden.reshape(NN, H).astype(F32)

    srcf = edges[0].astype(jnp.int32)[:, None]
    dstf = edges[1].astype(jnp.int32)[:, None]
    srcr = edges[0].astype(jnp.int32).reshape(e // 128, 128)
    dstr = edges[1].astype(jnp.int32).reshape(e // 128, 128)

    tab, wv, nnb, idxcat = pl.pallas_call(
        _prep_body,
        out_shape=(jax.ShapeDtypeStruct((2 * NN, M), F32),
                   jax.ShapeDtypeStruct((e, 1), F32),
                   jax.ShapeDtypeStruct((N, 1), F32),
                   jax.ShapeDtypeStruct((2 * ef // 128, 128), jnp.int32)),
    )(hf, srcf, dstf, srcr, dstr, Wm1)

    gs, gd = _sc_gather(tab, idxcat, ef)

    full = lambda shape: pl.BlockSpec(shape, lambda i: tuple(0 for _ in shape))
    co, ho = pl.pallas_call(
        _edge_body,
        grid=(B,),
        in_specs=[pl.BlockSpec((e, M), lambda i: (i, 0)),
                  pl.BlockSpec((e, M), lambda i: (i, 0)),
                  full((e, 1)), full((e, 1)),
                  pl.BlockSpec((N, 3), lambda i: (i, 0)),
                  pl.BlockSpec((N, H), lambda i: (i, 0)),
                  full((e, 1)), full((N, 1)),
                  full((1, M)), full((1, M)), full((M, M)), full((1, M)),
                  full((M, M)), full((1, M)), full((1, M)),
                  full((H, M)), full((M, M)), full((1, M)),
                  full((M, H)), full((1, H))],
        out_specs=(pl.BlockSpec((N, 3), lambda i: (i, 0)),
                   pl.BlockSpec((N, H), lambda i: (i, 0))),
        out_shape=(jax.ShapeDtypeStruct((NN, 3), F32),
                   jax.ShapeDtypeStruct((NN, H), F32)),
    )(gs, gd, srcf, dstf, cf, hf, wv, nnb, Wm1[0:1], bm1[None, :], Wm2,
      bm2[None, :], Wc1, bc1[None, :], Wc2.reshape(1, M),
      Wh1[:H], Wh1[H:], bh1[None, :], Wh2, bh2[None, :])

    coords_out = co.reshape(B, N, 3)
    hidden_out = ho.reshape(B, N, H)
    return coords_out, hidden_out
